# baseline (speedup 1.0000x reference)
import functools

import jax
import jax.numpy as jnp
from jax import lax
from jax.experimental import pallas as pl
from jax.experimental.pallas import tpu as pltpu
from jax.experimental.pallas import tpu_sc as plsc

N = 10000
E = 320000
NC = 2
NS = 16
NW = NC * NS
EPW = E // NW
CH = 80
NCH = EPW // CH
SG = 25
NSG = NCH // SG
SGR = SG * CH
NPS = N // NS
E4 = E // 4
E8 = E // 8

_MESH = plsc.VectorSubcoreMesh(core_axis_name="c", subcore_axis_name="s")
_SC_PARAMS = pltpu.CompilerParams(use_tc_tiling_on_sc=False)


def _make_gather(n_tab):
  @functools.partial(
      pl.kernel,
      out_type=[jax.ShapeDtypeStruct((E, 32), jnp.float32)] * n_tab,
      mesh=_MESH,
      compiler_params=_SC_PARAMS,
      scratch_types=(
          [pltpu.VMEM((NCH, CH), jnp.int32)] * n_tab
          + [pltpu.VMEM((SGR, 32), jnp.float32), pltpu.SemaphoreType.DMA]
      ),
  )
  def gather_kernel(*refs):
    tabs = refs[0:n_tab]
    idxs_hbm = refs[n_tab:2 * n_tab]
    outs = refs[2 * n_tab:3 * n_tab]
    idx_vs = refs[3 * n_tab:4 * n_tab]
    rows_v = refs[4 * n_tab]
    sem = refs[4 * n_tab + 1]

    cid = lax.axis_index("c")
    sid = lax.axis_index("s")
    wid = cid * NS + sid

    for t in range(n_tab):
      pltpu.sync_copy(idxs_hbm[t].at[wid], idx_vs[t])

    for t in range(n_tab):
      @pl.loop(0, NSG)
      def _super(s, t=t):
        @pl.loop(0, SG)
        def _fire(c):
          pltpu.async_copy(
              tabs[t].at[idx_vs[t].at[s * SG + c]],
              rows_v.at[pl.ds(c * CH, CH)],
              sem,
          )
        pltpu.make_async_copy(tabs[t].at[pl.ds(0, SGR)], rows_v, sem).wait()
        pltpu.sync_copy(rows_v, outs[t].at[pl.ds(wid * EPW + s * SGR, SGR)])

  return gather_kernel


_gather1 = _make_gather(1)
_gather2 = _make_gather(2)


@functools.partial(
    pl.kernel,
    out_type=[
        jax.ShapeDtypeStruct((NC, N, 32), jnp.float32),
        jax.ShapeDtypeStruct((NC, N, 16), jnp.float32),
    ],
    mesh=_MESH,
    compiler_params=_SC_PARAMS,
    scratch_types=[
        pltpu.VMEM((SG, CH), jnp.int32),
        pltpu.VMEM((SGR, 32), jnp.float32),
        pltpu.VMEM((SGR, 16), jnp.float32),
        pltpu.VMEM_SHARED((N, 32), jnp.float32),
        pltpu.VMEM_SHARED((N, 16), jnp.float32),
        pltpu.SemaphoreType.DMA,
    ],
)
def _scatter_add(m_hbm, dsti_hbm, sums_hbm, cnts_hbm,
                 idx_v, rows_v, ones_v, acc, acc_c, sem):
  cid = lax.axis_index("c")
  sid = lax.axis_index("s")
  wid = cid * NS + sid

  z16 = jnp.zeros((16,), jnp.float32)
  o16 = jnp.ones((16,), jnp.float32)

  @pl.loop(0, NPS)
  def _zero(i):
    rows_v[i, pl.ds(0, 16)] = z16
    rows_v[i, pl.ds(16, 16)] = z16
    ones_v[i, pl.ds(0, 16)] = z16

  pltpu.sync_copy(rows_v.at[pl.ds(0, NPS)], acc.at[pl.ds(sid * NPS, NPS)])
  pltpu.sync_copy(ones_v.at[pl.ds(0, NPS)], acc_c.at[pl.ds(sid * NPS, NPS)])

  @pl.loop(0, SGR)
  def _ones(i):
    ones_v[i, pl.ds(0, 16)] = o16

  plsc.subcore_barrier()

  @pl.loop(0, NSG)
  def _super(s):
    pltpu.sync_copy(dsti_hbm.at[wid, pl.ds(s * SG, SG)], idx_v)
    pltpu.sync_copy(m_hbm.at[pl.ds(wid * EPW + s * SGR, SGR)], rows_v)

    @pl.loop(0, SG)
    def _fire(c):
      irow = idx_v.at[c]
      pltpu.async_copy(rows_v.at[pl.ds(c * CH, CH)], acc.at[irow], sem,
                       add=True)
      pltpu.async_copy(ones_v.at[pl.ds(c * CH, CH)], acc_c.at[irow], sem,
                       add=True)
    pltpu.make_async_copy(m_hbm.at[pl.ds(0, SGR)], rows_v, sem).wait()
    pltpu.make_async_copy(m_hbm.at[pl.ds(0, SGR), pl.ds(0, 16)], ones_v,
                          sem).wait()

  plsc.subcore_barrier()
  sl = pl.ds(sid * NPS, NPS)
  pltpu.sync_copy(acc.at[sl], sums_hbm.at[cid, sl])
  pltpu.sync_copy(acc_c.at[sl], cnts_hbm.at[cid, sl])


def _node_pre_body(x_ref, mem_ref, w1x_ref, w1m_ref, b1_ref, o_ref):
  o_ref[...] = (
      jnp.dot(x_ref[...], w1x_ref[...], preferred_element_type=jnp.float32)
      + jnp.dot(mem_ref[...], w1m_ref[...], preferred_element_type=jnp.float32)
      + b1_ref[...]
  )


def _node_pre(x, mem, w1x, w1m, b1):
  bn = 2000
  return pl.pallas_call(
      _node_pre_body,
      grid=(N // bn,),
      in_specs=[
          pl.BlockSpec((bn, 128), lambda i: (i, 0)),
          pl.BlockSpec((bn, 32), lambda i: (i, 0)),
          pl.BlockSpec((128, 32), lambda i: (0, 0)),
          pl.BlockSpec((32, 32), lambda i: (0, 0)),
          pl.BlockSpec((1, 32), lambda i: (0, 0)),
      ],
      out_specs=pl.BlockSpec((bn, 32), lambda i: (i, 0)),
      out_shape=jax.ShapeDtypeStruct((N, 32), jnp.float32),
  )(x, mem, w1x, w1m, b1)


BE4 = 3200


def _edge_mlp_body(g_ref, ea_ref, w1e_ref, w2_ref, b2_ref, w3_ref, b3_ref,
                   o_ref):
  ew = jnp.dot(ea_ref[...], w1e_ref[...], preferred_element_type=jnp.float32)
  t = jnp.maximum(g_ref[...] + ew, 0.0)
  t = jnp.dot(t, w2_ref[...], preferred_element_type=jnp.float32) + b2_ref[...]
  t = jnp.maximum(t, 0.0)
  o_ref[...] = (
      jnp.dot(t, w3_ref[...], preferred_element_type=jnp.float32) + b3_ref[...]
  )


def _edge_mlp(g4, ea4p, w1e_bd, w2_bd, b2_4, w3_bd, b3_4):
  return pl.pallas_call(
      _edge_mlp_body,
      grid=(E4 // BE4,),
      in_specs=[
          pl.BlockSpec((BE4, 128), lambda i: (i, 0)),
          pl.BlockSpec((BE4, 128), lambda i: (i, 0)),
          pl.BlockSpec((128, 128), lambda i: (0, 0)),
          pl.BlockSpec((128, 128), lambda i: (0, 0)),
          pl.BlockSpec((1, 128), lambda i: (0, 0)),
          pl.BlockSpec((128, 128), lambda i: (0, 0)),
          pl.BlockSpec((1, 128), lambda i: (0, 0)),
      ],
      out_specs=pl.BlockSpec((BE4, 128), lambda i: (i, 0)),
      out_shape=jax.ShapeDtypeStruct((E4, 128), jnp.float32),
  )(g4, ea4p, w1e_bd, w2_bd, b2_4, w3_bd, b3_4)


def _node_upd_body(p0_ref, p1_ref, c0_ref, c1_ref, x_ref, mem_ref,
                   wux_ref, wum_ref, wua_ref, bu_ref,
                   wca_ref, wcb_ref, bc1_ref, ha_ref, hb_ref):
  cnt = jnp.maximum(c0_ref[...] + c1_ref[...], 1.0)
  agg = (p0_ref[...] + p1_ref[...]) / cnt
  h = (
      jnp.dot(x_ref[...], wux_ref[...], preferred_element_type=jnp.float32)
      + jnp.dot(mem_ref[...], wum_ref[...], preferred_element_type=jnp.float32)
      + jnp.dot(agg, wua_ref[...], preferred_element_type=jnp.float32)
      + bu_ref[...]
  )
  h = jnp.maximum(h, 0.0)
  ha_ref[...] = (
      jnp.dot(h, wca_ref[...], preferred_element_type=jnp.float32)
      + bc1_ref[...]
  )
  hb_ref[...] = jnp.dot(h, wcb_ref[...], preferred_element_type=jnp.float32)


def _node_upd(p0, p1, c0, c1, x, mem, wux, wum, wua, bu, wca, wcb, bc1):
  bn = 2000
  return pl.pallas_call(
      _node_upd_body,
      grid=(N // bn,),
      in_specs=[
          pl.BlockSpec((bn, 32), lambda i: (i, 0)),
          pl.BlockSpec((bn, 32), lambda i: (i, 0)),
          pl.BlockSpec((bn, 1), lambda i: (i, 0)),
          pl.BlockSpec((bn, 1), lambda i: (i, 0)),
          pl.BlockSpec((bn, 128), lambda i: (i, 0)),
          pl.BlockSpec((bn, 32), lambda i: (i, 0)),
          pl.BlockSpec((128, 32), lambda i: (0, 0)),
          pl.BlockSpec((32, 32), lambda i: (0, 0)),
          pl.BlockSpec((32, 32), lambda i: (0, 0)),
          pl.BlockSpec((1, 32), lambda i: (0, 0)),
          pl.BlockSpec((32, 32), lambda i: (0, 0)),
          pl.BlockSpec((32, 32), lambda i: (0, 0)),
          pl.BlockSpec((1, 32), lambda i: (0, 0)),
      ],
      out_specs=[
          pl.BlockSpec((bn, 32), lambda i: (i, 0)),
          pl.BlockSpec((bn, 32), lambda i: (i, 0)),
      ],
      out_shape=[
          jax.ShapeDtypeStruct((N, 32), jnp.float32),
          jax.ShapeDtypeStruct((N, 32), jnp.float32),
      ],
  )(p0, p1, c0, c1, x, mem, wux, wum, wua, bu, wca, wcb, bc1)


def _head_body(a_ref, b_ref, wc2t_ref, bc2_ref, o_ref):
  z = jnp.maximum(a_ref[...] + b_ref[...], 0.0)
  zt = lax.dot_general(
      wc2t_ref[...], z, (((1,), (1,)), ((), ())),
      preferred_element_type=jnp.float32,
  )
  o_ref[...] = jax.nn.sigmoid(zt + bc2_ref[0, 0])


def _head(a4, b4, wc2t_bd, bc2):
  bg4 = 3200
  return pl.pallas_call(
      _head_body,
      grid=(E4 // bg4,),
      in_specs=[
          pl.BlockSpec((bg4, 128), lambda i: (i, 0)),
          pl.BlockSpec((bg4, 128), lambda i: (i, 0)),
          pl.BlockSpec((4, 128), lambda i: (0, 0)),
          pl.BlockSpec((1, 1), lambda i: (0, 0)),
      ],
      out_specs=pl.BlockSpec((4, bg4), lambda i: (0, i)),
      out_shape=jax.ShapeDtypeStruct((4, E4), jnp.float32),
  )(a4, b4, wc2t_bd, bc2)


def kernel(x, edge_index, edge_attr, y, memory,
           W1, b1, W2, b2, W3, b3, Wu, bu, Wc1, bc1, Wc2, bc2):
  del y
  src_r = edge_index[0].reshape(NW, NCH, CH)
  dst_r = edge_index[1].reshape(NW, NCH, CH)
  ea4p = jnp.concatenate(
      [edge_attr.reshape(E4, 4, 16),
       jnp.zeros((E4, 4, 16), jnp.float32)], axis=2).reshape(E4, 128)

  eye4 = jnp.eye(4, dtype=jnp.float32)
  w1e_ext = jnp.concatenate([W1[160:176], jnp.zeros((16, 32), jnp.float32)])
  w1e_bd = jnp.kron(eye4, w1e_ext)
  w2_bd = jnp.kron(eye4, W2)
  w3_bd = jnp.kron(eye4, W3)
  b2_4 = jnp.tile(b2, 4).reshape(1, 128)
  b3_4 = jnp.tile(b3, 4).reshape(1, 128)
  wc2t_bd = jnp.kron(eye4, Wc2.reshape(1, 32))

  b1r = b1.reshape(1, 32)
  bur = bu.reshape(1, 32)
  bc1r = bc1.reshape(1, 32)
  bc2r = bc2.reshape(1, 1)

  node_pre = _node_pre(x, memory, W1[:128], W1[128:160], b1r)
  (g_src,) = _gather1(node_pre, src_r)
  m4 = _edge_mlp(g_src.reshape(E4, 128), ea4p,
                 w1e_bd, w2_bd, b2_4, w3_bd, b3_4)
  sums, cnts = _scatter_add(m4.reshape(E, 32), dst_r)
  ha, hb = _node_upd(
      sums[0], sums[1], cnts[0, :, 0:1], cnts[1, :, 0:1],
      x, memory, Wu[:128], Wu[128:160], Wu[160:192], bur,
      Wc1[:32], Wc1[32:64], bc1r,
  )
  a_src, b_dst = _gather2(ha, hb, src_r, dst_r)
  out_t = _head(a_src.reshape(E4, 128), b_dst.reshape(E4, 128), wc2t_bd, bc2r)
  return out_t.T.reshape(E, 1)


# speedup vs baseline: 1.0162x; 1.0162x over previous
import functools

import jax
import jax.numpy as jnp
from jax import lax
from jax.experimental import pallas as pl
from jax.experimental.pallas import tpu as pltpu
from jax.experimental.pallas import tpu_sc as plsc

N = 10000
E = 320000
NC = 2
NS = 16
NW = NC * NS
EPW = E // NW
CH = 80
NCH = EPW // CH
SG = 25
NSG = NCH // SG
SGR = SG * CH
NPS = N // NS
E4 = E // 4
E8 = E // 8

_MESH = plsc.VectorSubcoreMesh(core_axis_name="c", subcore_axis_name="s")
_SC_PARAMS = pltpu.CompilerParams(use_tc_tiling_on_sc=False)


def _make_gather(n_tab):
  @functools.partial(
      pl.kernel,
      out_type=[jax.ShapeDtypeStruct((E, 32), jnp.float32)] * n_tab,
      mesh=_MESH,
      compiler_params=_SC_PARAMS,
      scratch_types=(
          [pltpu.VMEM((NCH, CH), jnp.int32)] * n_tab
          + [pltpu.VMEM((SGR, 32), jnp.float32), pltpu.SemaphoreType.DMA]
      ),
  )
  def gather_kernel(*refs):
    tabs = refs[0:n_tab]
    idxs_hbm = refs[n_tab:2 * n_tab]
    outs = refs[2 * n_tab:3 * n_tab]
    idx_vs = refs[3 * n_tab:4 * n_tab]
    rows_v = refs[4 * n_tab]
    sem = refs[4 * n_tab + 1]

    cid = lax.axis_index("c")
    sid = lax.axis_index("s")
    wid = cid * NS + sid

    for t in range(n_tab):
      pltpu.sync_copy(idxs_hbm[t].at[wid], idx_vs[t])

    for t in range(n_tab):
      @pl.loop(0, NSG)
      def _super(s, t=t):
        @pl.loop(0, SG)
        def _fire(c):
          pltpu.async_copy(
              tabs[t].at[idx_vs[t].at[s * SG + c]],
              rows_v.at[pl.ds(c * CH, CH)],
              sem,
          )
        pltpu.make_async_copy(tabs[t].at[pl.ds(0, SGR)], rows_v, sem).wait()
        pltpu.sync_copy(rows_v, outs[t].at[pl.ds(wid * EPW + s * SGR, SGR)])

  return gather_kernel


_gather1 = _make_gather(1)
_gather2 = _make_gather(2)


@functools.partial(
    pl.kernel,
    out_type=[
        jax.ShapeDtypeStruct((NC, N, 32), jnp.float32),
        jax.ShapeDtypeStruct((NC, N, 16), jnp.float32),
    ],
    mesh=_MESH,
    compiler_params=_SC_PARAMS,
    scratch_types=[
        pltpu.VMEM((SG, CH), jnp.int32),
        pltpu.VMEM((SGR, 32), jnp.float32),
        pltpu.VMEM((SGR, 16), jnp.float32),
        pltpu.VMEM_SHARED((N, 32), jnp.float32),
        pltpu.VMEM_SHARED((N, 16), jnp.float32),
        pltpu.SemaphoreType.DMA,
    ],
)
def _scatter_add(m_hbm, dsti_hbm, sums_hbm, cnts_hbm,
                 idx_v, rows_v, ones_v, acc, acc_c, sem):
  cid = lax.axis_index("c")
  sid = lax.axis_index("s")
  wid = cid * NS + sid

  z16 = jnp.zeros((16,), jnp.float32)
  o16 = jnp.ones((16,), jnp.float32)

  @pl.loop(0, NPS)
  def _zero(i):
    rows_v[i, pl.ds(0, 16)] = z16
    rows_v[i, pl.ds(16, 16)] = z16
    ones_v[i, pl.ds(0, 16)] = z16

  pltpu.sync_copy(rows_v.at[pl.ds(0, NPS)], acc.at[pl.ds(sid * NPS, NPS)])
  pltpu.sync_copy(ones_v.at[pl.ds(0, NPS)], acc_c.at[pl.ds(sid * NPS, NPS)])

  @pl.loop(0, SGR)
  def _ones(i):
    ones_v[i, pl.ds(0, 16)] = o16

  plsc.subcore_barrier()

  @pl.loop(0, NSG)
  def _super(s):
    pltpu.sync_copy(dsti_hbm.at[wid, pl.ds(s * SG, SG)], idx_v)
    pltpu.sync_copy(m_hbm.at[pl.ds(wid * EPW + s * SGR, SGR)], rows_v)

    @pl.loop(0, SG)
    def _fire(c):
      irow = idx_v.at[c]
      pltpu.async_copy(rows_v.at[pl.ds(c * CH, CH)], acc.at[irow], sem,
                       add=True)
      pltpu.async_copy(ones_v.at[pl.ds(c * CH, CH)], acc_c.at[irow], sem,
                       add=True)
    pltpu.make_async_copy(m_hbm.at[pl.ds(0, SGR)], rows_v, sem).wait()
    pltpu.make_async_copy(m_hbm.at[pl.ds(0, SGR), pl.ds(0, 16)], ones_v,
                          sem).wait()

  plsc.subcore_barrier()
  sl = pl.ds(sid * NPS, NPS)
  pltpu.sync_copy(acc.at[sl], sums_hbm.at[cid, sl])
  pltpu.sync_copy(acc_c.at[sl], cnts_hbm.at[cid, sl])


def _node_pre_body(x_ref, mem_ref, w1x_ref, w1m_ref, b1_ref, o_ref):
  o_ref[...] = (
      jnp.dot(x_ref[...], w1x_ref[...], preferred_element_type=jnp.float32)
      + jnp.dot(mem_ref[...], w1m_ref[...], preferred_element_type=jnp.float32)
      + b1_ref[...]
  )


def _node_pre(x, mem, w1x, w1m, b1):
  bn = 2000
  return pl.pallas_call(
      _node_pre_body,
      grid=(N // bn,),
      in_specs=[
          pl.BlockSpec((bn, 128), lambda i: (i, 0)),
          pl.BlockSpec((bn, 32), lambda i: (i, 0)),
          pl.BlockSpec((128, 32), lambda i: (0, 0)),
          pl.BlockSpec((32, 32), lambda i: (0, 0)),
          pl.BlockSpec((1, 32), lambda i: (0, 0)),
      ],
      out_specs=pl.BlockSpec((bn, 32), lambda i: (i, 0)),
      out_shape=jax.ShapeDtypeStruct((N, 32), jnp.float32),
  )(x, mem, w1x, w1m, b1)


BE4 = 3200


def _edge_mlp_body(g_ref, ea_ref, w1e_ref, w2_ref, b2_ref, w3_ref, b3_ref,
                   o_ref):
  ew = jnp.dot(ea_ref[...], w1e_ref[...], preferred_element_type=jnp.float32)
  t = jnp.maximum(g_ref[...] + ew, 0.0)
  t = jnp.dot(t, w2_ref[...], preferred_element_type=jnp.float32) + b2_ref[...]
  t = jnp.maximum(t, 0.0)
  o_ref[...] = (
      jnp.dot(t, w3_ref[...], preferred_element_type=jnp.float32) + b3_ref[...]
  )


def _edge_mlp(g4, ea4p, w1e_bd, w2_bd, b2_4, w3_bd, b3_4):
  return pl.pallas_call(
      _edge_mlp_body,
      grid=(E4 // BE4,),
      in_specs=[
          pl.BlockSpec((BE4, 128), lambda i: (i, 0)),
          pl.BlockSpec((BE4, 128), lambda i: (i, 0)),
          pl.BlockSpec((128, 128), lambda i: (0, 0)),
          pl.BlockSpec((128, 128), lambda i: (0, 0)),
          pl.BlockSpec((1, 128), lambda i: (0, 0)),
          pl.BlockSpec((128, 128), lambda i: (0, 0)),
          pl.BlockSpec((1, 128), lambda i: (0, 0)),
      ],
      out_specs=pl.BlockSpec((BE4, 128), lambda i: (i, 0)),
      out_shape=jax.ShapeDtypeStruct((E4, 128), jnp.float32),
  )(g4, ea4p, w1e_bd, w2_bd, b2_4, w3_bd, b3_4)


def _node_upd_body(p0_ref, p1_ref, c0_ref, c1_ref, x_ref, mem_ref,
                   wux_ref, wum_ref, wua_ref, bu_ref,
                   wca_ref, wcb_ref, bc1_ref, ha_ref, hb_ref):
  cnt = jnp.maximum(c0_ref[...] + c1_ref[...], 1.0)
  agg = (p0_ref[...] + p1_ref[...]) / cnt
  h = (
      jnp.dot(x_ref[...], wux_ref[...], preferred_element_type=jnp.float32)
      + jnp.dot(mem_ref[...], wum_ref[...], preferred_element_type=jnp.float32)
      + jnp.dot(agg, wua_ref[...], preferred_element_type=jnp.float32)
      + bu_ref[...]
  )
  h = jnp.maximum(h, 0.0)
  ha_ref[...] = (
      jnp.dot(h, wca_ref[...], preferred_element_type=jnp.float32)
      + bc1_ref[...]
  )
  hb_ref[...] = jnp.dot(h, wcb_ref[...], preferred_element_type=jnp.float32)


def _node_upd(p0, p1, c0, c1, x, mem, wux, wum, wua, bu, wca, wcb, bc1):
  bn = 2000
  return pl.pallas_call(
      _node_upd_body,
      grid=(N // bn,),
      in_specs=[
          pl.BlockSpec((bn, 32), lambda i: (i, 0)),
          pl.BlockSpec((bn, 32), lambda i: (i, 0)),
          pl.BlockSpec((bn, 1), lambda i: (i, 0)),
          pl.BlockSpec((bn, 1), lambda i: (i, 0)),
          pl.BlockSpec((bn, 128), lambda i: (i, 0)),
          pl.BlockSpec((bn, 32), lambda i: (i, 0)),
          pl.BlockSpec((128, 32), lambda i: (0, 0)),
          pl.BlockSpec((32, 32), lambda i: (0, 0)),
          pl.BlockSpec((32, 32), lambda i: (0, 0)),
          pl.BlockSpec((1, 32), lambda i: (0, 0)),
          pl.BlockSpec((32, 32), lambda i: (0, 0)),
          pl.BlockSpec((32, 32), lambda i: (0, 0)),
          pl.BlockSpec((1, 32), lambda i: (0, 0)),
      ],
      out_specs=[
          pl.BlockSpec((bn, 32), lambda i: (i, 0)),
          pl.BlockSpec((bn, 32), lambda i: (i, 0)),
      ],
      out_shape=[
          jax.ShapeDtypeStruct((N, 32), jnp.float32),
          jax.ShapeDtypeStruct((N, 32), jnp.float32),
      ],
  )(p0, p1, c0, c1, x, mem, wux, wum, wua, bu, wca, wcb, bc1)


def _head_body(a_ref, b_ref, wc2_ref, bc2_ref, o_ref):
  z = jnp.maximum(a_ref[...] + b_ref[...], 0.0)
  o_ref[...] = jax.nn.sigmoid(
      jnp.dot(z, wc2_ref[...], preferred_element_type=jnp.float32)
      + bc2_ref[0, 0]
  )


def _head(a4, b4, wc2_bd, bc2):
  bg4 = 3200
  return pl.pallas_call(
      _head_body,
      grid=(E4 // bg4,),
      in_specs=[
          pl.BlockSpec((bg4, 128), lambda i: (i, 0)),
          pl.BlockSpec((bg4, 128), lambda i: (i, 0)),
          pl.BlockSpec((128, 4), lambda i: (0, 0)),
          pl.BlockSpec((1, 1), lambda i: (0, 0)),
      ],
      out_specs=pl.BlockSpec((bg4, 4), lambda i: (i, 0)),
      out_shape=jax.ShapeDtypeStruct((E4, 4), jnp.float32),
  )(a4, b4, wc2_bd, bc2)


def kernel(x, edge_index, edge_attr, y, memory,
           W1, b1, W2, b2, W3, b3, Wu, bu, Wc1, bc1, Wc2, bc2):
  del y
  src_r = edge_index[0].reshape(NW, NCH, CH)
  dst_r = edge_index[1].reshape(NW, NCH, CH)
  ea4p = jnp.concatenate(
      [edge_attr.reshape(E4, 4, 16),
       jnp.zeros((E4, 4, 16), jnp.float32)], axis=2).reshape(E4, 128)

  eye4 = jnp.eye(4, dtype=jnp.float32)
  w1e_ext = jnp.concatenate([W1[160:176], jnp.zeros((16, 32), jnp.float32)])
  w1e_bd = jnp.kron(eye4, w1e_ext)
  w2_bd = jnp.kron(eye4, W2)
  w3_bd = jnp.kron(eye4, W3)
  b2_4 = jnp.tile(b2, 4).reshape(1, 128)
  b3_4 = jnp.tile(b3, 4).reshape(1, 128)
  wc2_bd = jnp.kron(eye4, Wc2)

  b1r = b1.reshape(1, 32)
  bur = bu.reshape(1, 32)
  bc1r = bc1.reshape(1, 32)
  bc2r = bc2.reshape(1, 1)

  node_pre = _node_pre(x, memory, W1[:128], W1[128:160], b1r)
  (g_src,) = _gather1(node_pre, src_r)
  m4 = _edge_mlp(g_src.reshape(E4, 128), ea4p,
                 w1e_bd, w2_bd, b2_4, w3_bd, b3_4)
  sums, cnts = _scatter_add(m4.reshape(E, 32), dst_r)
  ha, hb = _node_upd(
      sums[0], sums[1], cnts[0, :, 0:1], cnts[1, :, 0:1],
      x, memory, Wu[:128], Wu[128:160], Wu[160:192], bur,
      Wc1[:32], Wc1[32:64], bc1r,
  )
  a_src, b_dst = _gather2(ha, hb, src_r, dst_r)
  out4 = _head(a_src.reshape(E4, 128), b_dst.reshape(E4, 128), wc2_bd, bc2r)
  return out4.reshape(E, 1)


# speedup vs baseline: 1.0163x; 1.0001x over previous
import functools

import jax
import jax.numpy as jnp
from jax import lax
from jax.experimental import pallas as pl
from jax.experimental.pallas import tpu as pltpu
from jax.experimental.pallas import tpu_sc as plsc

N = 10000
E = 320000
NC = 2
NS = 16
NW = NC * NS
EPW = E // NW
CH = 80
NCH = EPW // CH
SG = 25
NSG = NCH // SG
SGR = SG * CH
NPS = N // NS
E4 = E // 4

_MESH = plsc.VectorSubcoreMesh(core_axis_name="c", subcore_axis_name="s")
_SC_PARAMS = pltpu.CompilerParams(use_tc_tiling_on_sc=False)


def _make_gather(n_tab):
  @functools.partial(
      pl.kernel,
      out_type=[jax.ShapeDtypeStruct((E, 32), jnp.float32)] * n_tab,
      mesh=_MESH,
      compiler_params=_SC_PARAMS,
      scratch_types=(
          [pltpu.VMEM((NCH, CH), jnp.int32)] * n_tab
          + [pltpu.VMEM((SGR, 32), jnp.float32), pltpu.SemaphoreType.DMA]
      ),
  )
  def gather_kernel(*refs):
    tabs = refs[0:n_tab]
    idxs_hbm = refs[n_tab:2 * n_tab]
    outs = refs[2 * n_tab:3 * n_tab]
    idx_vs = refs[3 * n_tab:4 * n_tab]
    rows_v = refs[4 * n_tab]
    sem = refs[4 * n_tab + 1]

    cid = lax.axis_index("c")
    sid = lax.axis_index("s")
    wid = cid * NS + sid

    for t in range(n_tab):
      pltpu.sync_copy(idxs_hbm[t].at[wid], idx_vs[t])

    for t in range(n_tab):
      @pl.loop(0, NSG)
      def _super(s, t=t):
        @pl.loop(0, SG)
        def _fire(c):
          pltpu.async_copy(
              tabs[t].at[idx_vs[t].at[s * SG + c]],
              rows_v.at[pl.ds(c * CH, CH)],
              sem,
          )
        pltpu.make_async_copy(tabs[t].at[pl.ds(0, SGR)], rows_v, sem).wait()
        pltpu.sync_copy(rows_v, outs[t].at[pl.ds(wid * EPW + s * SGR, SGR)])

  return gather_kernel


_gather1 = _make_gather(1)
_gather2 = _make_gather(2)


@functools.partial(
    pl.kernel,
    out_type=[
        jax.ShapeDtypeStruct((NC, N, 32), jnp.float32),
        jax.ShapeDtypeStruct((NC, N, 16), jnp.float32),
    ],
    mesh=_MESH,
    compiler_params=_SC_PARAMS,
    scratch_types=[
        pltpu.VMEM((SG, CH), jnp.int32),
        pltpu.VMEM((SGR, 32), jnp.float32),
        pltpu.VMEM((SGR, 16), jnp.float32),
        pltpu.VMEM_SHARED((N, 32), jnp.float32),
        pltpu.VMEM_SHARED((N, 16), jnp.float32),
        pltpu.SemaphoreType.DMA,
    ],
)
def _scatter_add(m_hbm, dsti_hbm, sums_hbm, cnts_hbm,
                 idx_v, rows_v, ones_v, acc, acc_c, sem):
  cid = lax.axis_index("c")
  sid = lax.axis_index("s")
  wid = cid * NS + sid

  z16 = jnp.zeros((16,), jnp.float32)
  o16 = jnp.ones((16,), jnp.float32)

  @pl.loop(0, NPS)
  def _zero(i):
    rows_v[i, pl.ds(0, 16)] = z16
    rows_v[i, pl.ds(16, 16)] = z16
    ones_v[i, pl.ds(0, 16)] = z16

  pltpu.sync_copy(rows_v.at[pl.ds(0, NPS)], acc.at[pl.ds(sid * NPS, NPS)])
  pltpu.sync_copy(ones_v.at[pl.ds(0, NPS)], acc_c.at[pl.ds(sid * NPS, NPS)])

  @pl.loop(0, SGR)
  def _ones(i):
    ones_v[i, pl.ds(0, 16)] = o16

  plsc.subcore_barrier()

  @pl.loop(0, NSG)
  def _super(s):
    pltpu.sync_copy(dsti_hbm.at[wid, pl.ds(s * SG, SG)], idx_v)
    pltpu.sync_copy(m_hbm.at[pl.ds(wid * EPW + s * SGR, SGR)], rows_v)

    @pl.loop(0, SG)
    def _fire(c):
      irow = idx_v.at[c]
      pltpu.async_copy(rows_v.at[pl.ds(c * CH, CH)], acc.at[irow], sem,
                       add=True)
      pltpu.async_copy(ones_v.at[pl.ds(c * CH, CH)], acc_c.at[irow], sem,
                       add=True)
    pltpu.make_async_copy(m_hbm.at[pl.ds(0, SGR)], rows_v, sem).wait()
    pltpu.make_async_copy(m_hbm.at[pl.ds(0, SGR), pl.ds(0, 16)], ones_v,
                          sem).wait()

  plsc.subcore_barrier()
  sl = pl.ds(sid * NPS, NPS)
  pltpu.sync_copy(acc.at[sl], sums_hbm.at[cid, sl])
  pltpu.sync_copy(acc_c.at[sl], cnts_hbm.at[cid, sl])


def _node_pre_body(x_ref, mem_ref, w1x_ref, w1m_ref, b1_ref, o_ref):
  o_ref[...] = (
      jnp.dot(x_ref[...], w1x_ref[...], preferred_element_type=jnp.float32)
      + jnp.dot(mem_ref[...], w1m_ref[...], preferred_element_type=jnp.float32)
      + b1_ref[...]
  )


def _node_pre(x, mem, w1x, w1m, b1):
  bn = 2000
  return pl.pallas_call(
      _node_pre_body,
      grid=(N // bn,),
      in_specs=[
          pl.BlockSpec((bn, 128), lambda i: (i, 0)),
          pl.BlockSpec((bn, 32), lambda i: (i, 0)),
          pl.BlockSpec((128, 32), lambda i: (0, 0)),
          pl.BlockSpec((32, 32), lambda i: (0, 0)),
          pl.BlockSpec((1, 32), lambda i: (0, 0)),
      ],
      out_specs=pl.BlockSpec((bn, 32), lambda i: (i, 0)),
      out_shape=jax.ShapeDtypeStruct((N, 32), jnp.float32),
  )(x, mem, w1x, w1m, b1)


BE4 = 3200


def _edge_mlp_body(g_ref, ea_ref, w1e_ref, w2_ref, b2_ref, w3_ref, b3_ref,
                   o_ref):
  ew = jnp.dot(ea_ref[...], w1e_ref[...], preferred_element_type=jnp.float32)
  t = jnp.maximum(g_ref[...] + ew, 0.0)
  t = jnp.dot(t, w2_ref[...], preferred_element_type=jnp.float32) + b2_ref[...]
  t = jnp.maximum(t, 0.0)
  o_ref[...] = (
      jnp.dot(t, w3_ref[...], preferred_element_type=jnp.float32) + b3_ref[...]
  )


def _edge_mlp(g4, ea4p, w1e_bd, w2_bd, b2_4, w3_bd, b3_4):
  return pl.pallas_call(
      _edge_mlp_body,
      grid=(E4 // BE4,),
      in_specs=[
          pl.BlockSpec((BE4, 128), lambda i: (i, 0)),
          pl.BlockSpec((BE4, 128), lambda i: (i, 0)),
          pl.BlockSpec((128, 128), lambda i: (0, 0)),
          pl.BlockSpec((128, 128), lambda i: (0, 0)),
          pl.BlockSpec((1, 128), lambda i: (0, 0)),
          pl.BlockSpec((128, 128), lambda i: (0, 0)),
          pl.BlockSpec((1, 128), lambda i: (0, 0)),
      ],
      out_specs=pl.BlockSpec((BE4, 128), lambda i: (i, 0)),
      out_shape=jax.ShapeDtypeStruct((E4, 128), jnp.float32),
  )(g4, ea4p, w1e_bd, w2_bd, b2_4, w3_bd, b3_4)


def _node_upd_body(p0_ref, p1_ref, c0_ref, c1_ref, x_ref, mem_ref,
                   wux_ref, wum_ref, wua_ref, bu_ref,
                   wca_ref, wcb_ref, bc1_ref, ha_ref, hb_ref):
  cnt = jnp.maximum(c0_ref[...] + c1_ref[...], 1.0)
  agg = (p0_ref[...] + p1_ref[...]) / cnt
  h = (
      jnp.dot(x_ref[...], wux_ref[...], preferred_element_type=jnp.float32)
      + jnp.dot(mem_ref[...], wum_ref[...], preferred_element_type=jnp.float32)
      + jnp.dot(agg, wua_ref[...], preferred_element_type=jnp.float32)
      + bu_ref[...]
  )
  h = jnp.maximum(h, 0.0)
  ha_ref[...] = (
      jnp.dot(h, wca_ref[...], preferred_element_type=jnp.float32)
      + bc1_ref[...]
  )
  hb_ref[...] = jnp.dot(h, wcb_ref[...], preferred_element_type=jnp.float32)


def _node_upd(p0, p1, c0, c1, x, mem, wux, wum, wua, bu, wca, wcb, bc1):
  bn = 2000
  return pl.pallas_call(
      _node_upd_body,
      grid=(N // bn,),
      in_specs=[
          pl.BlockSpec((bn, 32), lambda i: (i, 0)),
          pl.BlockSpec((bn, 32), lambda i: (i, 0)),
          pl.BlockSpec((bn, 1), lambda i: (i, 0)),
          pl.BlockSpec((bn, 1), lambda i: (i, 0)),
          pl.BlockSpec((bn, 128), lambda i: (i, 0)),
          pl.BlockSpec((bn, 32), lambda i: (i, 0)),
          pl.BlockSpec((128, 32), lambda i: (0, 0)),
          pl.BlockSpec((32, 32), lambda i: (0, 0)),
          pl.BlockSpec((32, 32), lambda i: (0, 0)),
          pl.BlockSpec((1, 32), lambda i: (0, 0)),
          pl.BlockSpec((32, 32), lambda i: (0, 0)),
          pl.BlockSpec((32, 32), lambda i: (0, 0)),
          pl.BlockSpec((1, 32), lambda i: (0, 0)),
      ],
      out_specs=[
          pl.BlockSpec((bn, 32), lambda i: (i, 0)),
          pl.BlockSpec((bn, 32), lambda i: (i, 0)),
      ],
      out_shape=[
          jax.ShapeDtypeStruct((N, 32), jnp.float32),
          jax.ShapeDtypeStruct((N, 32), jnp.float32),
      ],
  )(p0, p1, c0, c1, x, mem, wux, wum, wua, bu, wca, wcb, bc1)


def _head_body(a_ref, b_ref, wc2_ref, bc2_ref, o_ref):
  z = jnp.maximum(a_ref[...] + b_ref[...], 0.0)
  o_ref[...] = jax.nn.sigmoid(
      jnp.dot(z, wc2_ref[...], preferred_element_type=jnp.float32)
      + bc2_ref[0, 0]
  )


def _head(a4, b4, wc2_bd, bc2):
  bg4 = 3200
  return pl.pallas_call(
      _head_body,
      grid=(E4 // bg4,),
      in_specs=[
          pl.BlockSpec((bg4, 128), lambda i: (i, 0)),
          pl.BlockSpec((bg4, 128), lambda i: (i, 0)),
          pl.BlockSpec((128, 4), lambda i: (0, 0)),
          pl.BlockSpec((1, 1), lambda i: (0, 0)),
      ],
      out_specs=pl.BlockSpec((bg4, 4), lambda i: (i, 0)),
      out_shape=jax.ShapeDtypeStruct((E4, 4), jnp.float32),
  )(a4, b4, wc2_bd, bc2)


def kernel(x, edge_index, edge_attr, y, memory,
           W1, b1, W2, b2, W3, b3, Wu, bu, Wc1, bc1, Wc2, bc2):
  del y
  src_r = edge_index[0].reshape(NW, NCH, CH)
  dst_r = edge_index[1].reshape(NW, NCH, CH)
  ea4p = jnp.concatenate(
      [edge_attr.reshape(E4, 4, 16),
       jnp.zeros((E4, 4, 16), jnp.float32)], axis=2).reshape(E4, 128)

  eye4 = jnp.eye(4, dtype=jnp.float32)
  w1e_ext = jnp.concatenate([W1[160:176], jnp.zeros((16, 32), jnp.float32)])
  w1e_bd = jnp.kron(eye4, w1e_ext)
  w2_bd = jnp.kron(eye4, W2)
  w3_bd = jnp.kron(eye4, W3)
  b2_4 = jnp.tile(b2, 4).reshape(1, 128)
  b3_4 = jnp.tile(b3, 4).reshape(1, 128)
  wc2_bd = jnp.kron(eye4, Wc2)

  b1r = b1.reshape(1, 32)
  bur = bu.reshape(1, 32)
  bc1r = bc1.reshape(1, 32)
  bc2r = bc2.reshape(1, 1)

  node_pre = _node_pre(x, memory, W1[:128], W1[128:160], b1r)
  (g_src,) = _gather1(node_pre, src_r)
  m4 = _edge_mlp(g_src.reshape(E4, 128), ea4p,
                 w1e_bd, w2_bd, b2_4, w3_bd, b3_4)
  sums, cnts = _scatter_add(m4.reshape(E, 32), dst_r)
  ha, hb = _node_upd(
      sums[0], sums[1], cnts[0, :, 0:1], cnts[1, :, 0:1],
      x, memory, Wu[:128], Wu[128:160], Wu[160:192], bur,
      Wc1[:32], Wc1[32:64], bc1r,
  )
  a_src, b_dst = _gather2(ha, hb, src_r, dst_r)
  out4 = _head(a_src.reshape(E4, 128), b_dst.reshape(E4, 128), wc2_bd, bc2r)
  return out4.reshape(E, 1)
